# baseline (speedup 1.0000x reference)
import functools
import math

import jax
import jax.numpy as jnp
from jax import lax
from jax.experimental import pallas as pl
from jax.experimental.pallas import tpu as pltpu
from jax.experimental.pallas import tpu_sc as plsc

EMB_D = 64
SCALE = math.sqrt(EMB_D)

NUM_CORES = 2
NUM_SUBCORES = 16
NW = NUM_CORES * NUM_SUBCORES

CHUNK = 256
SEQ = 200
BATCH = 4096
BLKS_PER_S = BATCH // CHUNK
N_CHUNKS = SEQ * BLKS_PER_S
CPW = N_CHUNKS // NW


def _chunk_pos(c):
    return c // BLKS_PER_S, (c % BLKS_PER_S) * CHUNK


def _emb_body(tok_hbm, table_hbm, out_hbm,
              tok0, tok1, idx0, idx1, rows0, rows1, outb0, outb1,
              tsem0, tsem1, gsem0, gsem1, osem0, osem1):
    w = lax.axis_index("s") * NUM_CORES + lax.axis_index("c")
    c0 = w * CPW

    toks = (tok0, tok1)
    idxs = (idx0, idx1)
    rows = (rows0, rows1)
    outs = (outb0, outb1)
    tsems = (tsem0, tsem1)
    gsems = (gsem0, gsem1)
    osems = (osem0, osem1)

    def tok_copy(g, b):
        s, b0 = _chunk_pos(c0 + g)
        return pltpu.make_async_copy(
            tok_hbm.at[s, pl.ds(b0, CHUNK)], toks[b], tsems[b])

    def gather_copy(b, j):
        return pltpu.make_async_copy(
            table_hbm.at[idxs[b].at[j]],
            rows[b].at[pl.ds(j * 128, 128)], gsems[b])

    def out_copy(g, b):
        s, b0 = _chunk_pos(c0 + g)
        return pltpu.make_async_copy(
            outs[b], out_hbm.at[s, :, pl.ds(b0, CHUNK)], osems[b])

    def compute_idx(b):
        for j in range(2):
            for i in range(8):
                v = toks[b][pl.ds(j * 128 + i * 16, 16)]
                idxs[b][j, pl.ds(i * 16, 16)] = lax.shift_right_logical(v, 1)

    def transpose_scale(b):
        iota = lax.iota(jnp.int32, 16)

        def blk(k, carry):
            tokv = toks[b][pl.ds(k * 16, 16)]
            colb = lax.shift_left(jnp.bitwise_and(tokv, 1), 6)
            ridx = iota + k * 16
            for d in range(EMB_D):
                v = plsc.load_gather(rows[b], [ridx, colb + d])
                outs[b][d, pl.ds(k * 16, 16)] = v * SCALE
            return carry

        lax.fori_loop(0, CHUNK // 16, blk, 0)

    cp = tok_copy(0, 0)
    cp.start()
    cp.wait()
    compute_idx(0)
    for j in range(2):
        gather_copy(0, j).start()
    tok_copy(1, 1).start()

    def iter_g(g, b):
        nb = 1 - b

        @pl.when(g >= 2)
        def _():
            out_copy(g, b).wait()

        for j in range(2):
            gather_copy(b, j).wait()
        transpose_scale(b)
        out_copy(g, b).start()

        @pl.when(g + 1 < CPW)
        def _():
            tok_copy(g + 1, nb).wait()
            compute_idx(nb)
            for j in range(2):
                gather_copy(nb, j).start()

        @pl.when(g + 2 < CPW)
        def _():
            tok_copy(g + 2, b).start()

    def pair(i, carry):
        iter_g(2 * i, 0)
        iter_g(2 * i + 1, 1)
        return carry

    lax.fori_loop(0, CPW // 2, pair, 0)

    out_copy(CPW - 2, 0).wait()
    out_copy(CPW - 1, 1).wait()


def kernel(tokens, embedding_weight):
    bt, seq = tokens.shape
    vocab = embedding_weight.shape[0]
    tok_t = tokens.T
    table2 = embedding_weight.reshape(vocab // 2, 128)

    mesh = plsc.VectorSubcoreMesh(core_axis_name="c", subcore_axis_name="s")
    emb = pl.kernel(
        _emb_body,
        mesh=mesh,
        out_type=jax.ShapeDtypeStruct((seq, EMB_D, bt), jnp.float32),
        scratch_types=[
            pltpu.VMEM((CHUNK,), jnp.int32),
            pltpu.VMEM((CHUNK,), jnp.int32),
            pltpu.VMEM((2, 128), jnp.int32),
            pltpu.VMEM((2, 128), jnp.int32),
            pltpu.VMEM((CHUNK, 128), jnp.float32),
            pltpu.VMEM((CHUNK, 128), jnp.float32),
            pltpu.VMEM((EMB_D, CHUNK), jnp.float32),
            pltpu.VMEM((EMB_D, CHUNK), jnp.float32),
            pltpu.SemaphoreType.DMA,
            pltpu.SemaphoreType.DMA,
            pltpu.SemaphoreType.DMA,
            pltpu.SemaphoreType.DMA,
            pltpu.SemaphoreType.DMA,
            pltpu.SemaphoreType.DMA,
        ],
        compiler_params=pltpu.CompilerParams(
            use_tc_tiling_on_sc=True, needs_layout_passes=False),
    )

    out = emb(tok_t, table2)
    return jnp.transpose(out, (2, 0, 1))


# speedup vs baseline: 1.3099x; 1.3099x over previous
import functools
import math

import jax
import jax.numpy as jnp
from jax import lax
from jax.experimental import pallas as pl
from jax.experimental.pallas import tpu as pltpu
from jax.experimental.pallas import tpu_sc as plsc

EMB_D = 64
SCALE = math.sqrt(EMB_D)

NUM_CORES = 2
NUM_SUBCORES = 16
NW = NUM_CORES * NUM_SUBCORES

CHUNK = 256
SEQ = 200
BATCH = 4096
BLKS_PER_S = BATCH // CHUNK
N_CHUNKS = SEQ * BLKS_PER_S
CPW = N_CHUNKS // NW


def _chunk_pos(c):
    return c // BLKS_PER_S, (c % BLKS_PER_S) * CHUNK


def _emb_body(tok_hbm, table_hbm, out_hbm,
              tok0, tok1, idx0, idx1, rows0, rows1, outb0, outb1,
              tsem0, tsem1, gsem0, gsem1, osem0, osem1):
    w = lax.axis_index("s") * NUM_CORES + lax.axis_index("c")
    c0 = w * CPW

    toks = (tok0, tok1)
    idxs = (idx0, idx1)
    rows = (rows0, rows1)
    outs = (outb0, outb1)
    tsems = (tsem0, tsem1)
    gsems = (gsem0, gsem1)
    osems = (osem0, osem1)

    def tok_copy(g, b):
        s, b0 = _chunk_pos(c0 + g)
        return pltpu.make_async_copy(
            tok_hbm.at[s, pl.ds(b0, CHUNK)], toks[b], tsems[b])

    def gather_copy(b, j):
        return pltpu.make_async_copy(
            table_hbm.at[idxs[b].at[j]],
            rows[b].at[pl.ds(j * 128, 128)], gsems[b])

    def out_copy(g, b):
        s, b0 = _chunk_pos(c0 + g)
        return pltpu.make_async_copy(
            outs[b], out_hbm.at[s, :, pl.ds(b0, CHUNK)], osems[b])

    def compute_idx(b):
        for j in range(2):
            for i in range(8):
                v = toks[b][pl.ds(j * 128 + i * 16, 16)]
                idxs[b][j, pl.ds(i * 16, 16)] = lax.shift_right_logical(v, 1)

    def transpose_scale(b):
        iota = lax.iota(jnp.int32, 16)

        @plsc.parallel_loop(0, CHUNK // 16, unroll=2)
        def _(k):
            tokv = toks[b][pl.ds(k * 16, 16)]
            colb = lax.shift_left(jnp.bitwise_and(tokv, 1), 6)
            ridx = iota + k * 16
            for d in range(EMB_D):
                v = plsc.load_gather(rows[b], [ridx, colb + d])
                outs[b][d, pl.ds(k * 16, 16)] = v * SCALE

    cp = tok_copy(0, 0)
    cp.start()
    cp.wait()
    compute_idx(0)
    for j in range(2):
        gather_copy(0, j).start()
    tok_copy(1, 1).start()

    def iter_g(g, b):
        nb = 1 - b

        @pl.when(g >= 2)
        def _():
            out_copy(g, b).wait()

        for j in range(2):
            gather_copy(b, j).wait()
        transpose_scale(b)
        out_copy(g, b).start()

        @pl.when(g + 1 < CPW)
        def _():
            tok_copy(g + 1, nb).wait()
            compute_idx(nb)
            for j in range(2):
                gather_copy(nb, j).start()

        @pl.when(g + 2 < CPW)
        def _():
            tok_copy(g + 2, b).start()

    def pair(i, carry):
        iter_g(2 * i, 0)
        iter_g(2 * i + 1, 1)
        return carry

    lax.fori_loop(0, CPW // 2, pair, 0)

    out_copy(CPW - 2, 0).wait()
    out_copy(CPW - 1, 1).wait()


def kernel(tokens, embedding_weight):
    bt, seq = tokens.shape
    vocab = embedding_weight.shape[0]
    tok_t = tokens.T
    table2 = embedding_weight.reshape(vocab // 2, 128)

    mesh = plsc.VectorSubcoreMesh(core_axis_name="c", subcore_axis_name="s")
    emb = pl.kernel(
        _emb_body,
        mesh=mesh,
        out_type=jax.ShapeDtypeStruct((seq, EMB_D, bt), jnp.float32),
        scratch_types=[
            pltpu.VMEM((CHUNK,), jnp.int32),
            pltpu.VMEM((CHUNK,), jnp.int32),
            pltpu.VMEM((2, 128), jnp.int32),
            pltpu.VMEM((2, 128), jnp.int32),
            pltpu.VMEM((CHUNK, 128), jnp.float32),
            pltpu.VMEM((CHUNK, 128), jnp.float32),
            pltpu.VMEM((EMB_D, CHUNK), jnp.float32),
            pltpu.VMEM((EMB_D, CHUNK), jnp.float32),
            pltpu.SemaphoreType.DMA,
            pltpu.SemaphoreType.DMA,
            pltpu.SemaphoreType.DMA,
            pltpu.SemaphoreType.DMA,
            pltpu.SemaphoreType.DMA,
            pltpu.SemaphoreType.DMA,
        ],
        compiler_params=pltpu.CompilerParams(
            use_tc_tiling_on_sc=True, needs_layout_passes=False),
    )

    out = emb(tok_t, table2)
    return jnp.transpose(out, (2, 0, 1))


# speedup vs baseline: 1.3977x; 1.0670x over previous
import functools
import math

import jax
import jax.numpy as jnp
from jax import lax
from jax.experimental import pallas as pl
from jax.experimental.pallas import tpu as pltpu
from jax.experimental.pallas import tpu_sc as plsc

EMB_D = 64
SCALE = math.sqrt(EMB_D)

NUM_CORES = 2
NUM_SUBCORES = 16
NW = NUM_CORES * NUM_SUBCORES

CHUNK = 256
SEQ = 200
BATCH = 4096
BLKS_PER_S = BATCH // CHUNK
N_CHUNKS = SEQ * BLKS_PER_S
CPW = N_CHUNKS // NW


def _chunk_pos(c):
    return c // BLKS_PER_S, (c % BLKS_PER_S) * CHUNK


def _emb_body(tok_hbm, table_hbm, out_hbm,
              tok0, tok1, idx0, idx1, rows0, rows1, outb0, outb1,
              tsem0, tsem1, gsem0, gsem1, osem0, osem1):
    w = lax.axis_index("s") * NUM_CORES + lax.axis_index("c")
    c0 = w * CPW

    toks = (tok0, tok1)
    idxs = (idx0, idx1)
    rows = (rows0, rows1)
    outs = (outb0, outb1)
    tsems = (tsem0, tsem1)
    gsems = (gsem0, gsem1)
    osems = (osem0, osem1)

    def tok_copy(g, b):
        s, b0 = _chunk_pos(c0 + g)
        return pltpu.make_async_copy(
            tok_hbm.at[s, pl.ds(b0, CHUNK)], toks[b], tsems[b])

    def gather_copy(b, j):
        return pltpu.make_async_copy(
            table_hbm.at[idxs[b].at[j]],
            rows[b].at[pl.ds(j * 128, 128)], gsems[b])

    def out_copy(g, b):
        s, b0 = _chunk_pos(c0 + g)
        return pltpu.make_async_copy(
            outs[b], out_hbm.at[s, :, pl.ds(b0, CHUNK)], osems[b])

    def compute_idx(b):
        for j in range(2):
            for i in range(8):
                v = toks[b][pl.ds(j * 128 + i * 16, 16)]
                idxs[b][j, pl.ds(i * 16, 16)] = lax.shift_right_logical(v, 1)

    def transpose_scale(b):
        iota = lax.iota(jnp.int32, 16)

        @plsc.parallel_loop(0, CHUNK // 16, unroll=2)
        def _(k):
            tokv = toks[b][pl.ds(k * 16, 16)]
            colb = lax.shift_left(jnp.bitwise_and(tokv, 1), 6)
            ridx = iota + k * 16
            for j in range(EMB_D):
                dvec = jnp.bitwise_and(iota + j, EMB_D - 1)
                v = plsc.load_gather(rows[b], [ridx, colb + dvec])
                plsc.store_scatter(outs[b], [dvec, ridx], v * SCALE)

    cp = tok_copy(0, 0)
    cp.start()
    cp.wait()
    compute_idx(0)
    for j in range(2):
        gather_copy(0, j).start()
    tok_copy(1, 1).start()

    def iter_g(g, b):
        nb = 1 - b

        @pl.when(g >= 2)
        def _():
            out_copy(g, b).wait()

        for j in range(2):
            gather_copy(b, j).wait()
        transpose_scale(b)
        out_copy(g, b).start()

        @pl.when(g + 1 < CPW)
        def _():
            tok_copy(g + 1, nb).wait()
            compute_idx(nb)
            for j in range(2):
                gather_copy(nb, j).start()

        @pl.when(g + 2 < CPW)
        def _():
            tok_copy(g + 2, b).start()

    def pair(i, carry):
        iter_g(2 * i, 0)
        iter_g(2 * i + 1, 1)
        return carry

    lax.fori_loop(0, CPW // 2, pair, 0)

    out_copy(CPW - 2, 0).wait()
    out_copy(CPW - 1, 1).wait()


def kernel(tokens, embedding_weight):
    bt, seq = tokens.shape
    vocab = embedding_weight.shape[0]
    tok_t = tokens.T
    table2 = embedding_weight.reshape(vocab // 2, 128)

    mesh = plsc.VectorSubcoreMesh(core_axis_name="c", subcore_axis_name="s")
    emb = pl.kernel(
        _emb_body,
        mesh=mesh,
        out_type=jax.ShapeDtypeStruct((seq, EMB_D, bt), jnp.float32),
        scratch_types=[
            pltpu.VMEM((CHUNK,), jnp.int32),
            pltpu.VMEM((CHUNK,), jnp.int32),
            pltpu.VMEM((2, 128), jnp.int32),
            pltpu.VMEM((2, 128), jnp.int32),
            pltpu.VMEM((CHUNK, 128), jnp.float32),
            pltpu.VMEM((CHUNK, 128), jnp.float32),
            pltpu.VMEM((EMB_D, CHUNK), jnp.float32),
            pltpu.VMEM((EMB_D, CHUNK), jnp.float32),
            pltpu.SemaphoreType.DMA,
            pltpu.SemaphoreType.DMA,
            pltpu.SemaphoreType.DMA,
            pltpu.SemaphoreType.DMA,
            pltpu.SemaphoreType.DMA,
            pltpu.SemaphoreType.DMA,
        ],
        compiler_params=pltpu.CompilerParams(
            use_tc_tiling_on_sc=True, needs_layout_passes=False,
            disable_bounds_checks=True),
    )

    out = emb(tok_t, table2)
    return jnp.transpose(out, (2, 0, 1))


# speedup vs baseline: 1.8375x; 1.3147x over previous
import functools
import math

import jax
import jax.numpy as jnp
from jax import lax
from jax.experimental import pallas as pl
from jax.experimental.pallas import tpu as pltpu
from jax.experimental.pallas import tpu_sc as plsc

EMB_D = 64
SCALE = math.sqrt(EMB_D)

NUM_CORES = 2
NUM_SUBCORES = 16
NW = NUM_CORES * NUM_SUBCORES

CHUNK = 256
SEQ = 200
BATCH = 4096
BLKS_PER_S = BATCH // CHUNK
N_CHUNKS = SEQ * BLKS_PER_S
CPW = N_CHUNKS // NW


def _chunk_pos(c):
    return c // BLKS_PER_S, (c % BLKS_PER_S) * CHUNK


def _emb_body(tok_hbm, table_hbm, out_hbm,
              tok0, tok1, idx0, idx1, rows0, rows1, outb0, outb1,
              tsem0, tsem1, gsem0, gsem1, osem0, osem1):
    w = lax.axis_index("s") * NUM_CORES + lax.axis_index("c")
    c0 = w * CPW

    toks = (tok0, tok1)
    idxs = (idx0, idx1)
    rows = (rows0, rows1)
    outs = (outb0, outb1)
    tsems = (tsem0, tsem1)
    gsems = (gsem0, gsem1)
    osems = (osem0, osem1)

    def tok_copy(g, b):
        s, b0 = _chunk_pos(c0 + g)
        return pltpu.make_async_copy(
            tok_hbm.at[s, pl.ds(b0, CHUNK)], toks[b], tsems[b])

    def gather_copy(b, j):
        return pltpu.make_async_copy(
            table_hbm.at[idxs[b].at[j]],
            rows[b].at[pl.ds(j * 128, 128)], gsems[b])

    def out_copy(g, b):
        s, b0 = _chunk_pos(c0 + g)
        return pltpu.make_async_copy(
            outs[b], out_hbm.at[s, :, pl.ds(b0, CHUNK)], osems[b])

    def compute_idx(b):
        for j in range(2):
            for i in range(8):
                v = toks[b][pl.ds(j * 128 + i * 16, 16)]
                idxs[b][j, pl.ds(i * 16, 16)] = lax.shift_right_logical(v, 1)

    def transpose_scale(b):
        iota = lax.iota(jnp.int32, 16)

        @plsc.parallel_loop(0, CHUNK // 16, unroll=8)
        def _(k):
            tokv = toks[b][pl.ds(k * 16, 16)]
            colb = lax.shift_left(jnp.bitwise_and(tokv, 1), 6)
            ridx = iota + k * 16
            for j in range(EMB_D):
                dvec = jnp.bitwise_and(iota + j, EMB_D - 1)
                v = plsc.load_gather(rows[b], [ridx, colb + dvec])
                plsc.store_scatter(outs[b], [dvec, ridx], v * SCALE)

    cp = tok_copy(0, 0)
    cp.start()
    cp.wait()
    compute_idx(0)
    for j in range(2):
        gather_copy(0, j).start()
    tok_copy(1, 1).start()

    def iter_g(g, b):
        nb = 1 - b

        @pl.when(g >= 2)
        def _():
            out_copy(g, b).wait()

        for j in range(2):
            gather_copy(b, j).wait()
        transpose_scale(b)
        out_copy(g, b).start()

        @pl.when(g + 1 < CPW)
        def _():
            tok_copy(g + 1, nb).wait()
            compute_idx(nb)
            for j in range(2):
                gather_copy(nb, j).start()

        @pl.when(g + 2 < CPW)
        def _():
            tok_copy(g + 2, b).start()

    def pair(i, carry):
        iter_g(2 * i, 0)
        iter_g(2 * i + 1, 1)
        return carry

    lax.fori_loop(0, CPW // 2, pair, 0)

    out_copy(CPW - 2, 0).wait()
    out_copy(CPW - 1, 1).wait()


def kernel(tokens, embedding_weight):
    bt, seq = tokens.shape
    vocab = embedding_weight.shape[0]
    tok_t = tokens.T
    table2 = embedding_weight.reshape(vocab // 2, 128)

    mesh = plsc.VectorSubcoreMesh(core_axis_name="c", subcore_axis_name="s")
    emb = pl.kernel(
        _emb_body,
        mesh=mesh,
        out_type=jax.ShapeDtypeStruct((seq, EMB_D, bt), jnp.float32),
        scratch_types=[
            pltpu.VMEM((CHUNK,), jnp.int32),
            pltpu.VMEM((CHUNK,), jnp.int32),
            pltpu.VMEM((2, 128), jnp.int32),
            pltpu.VMEM((2, 128), jnp.int32),
            pltpu.VMEM((CHUNK, 128), jnp.float32),
            pltpu.VMEM((CHUNK, 128), jnp.float32),
            pltpu.VMEM((EMB_D, CHUNK), jnp.float32),
            pltpu.VMEM((EMB_D, CHUNK), jnp.float32),
            pltpu.SemaphoreType.DMA,
            pltpu.SemaphoreType.DMA,
            pltpu.SemaphoreType.DMA,
            pltpu.SemaphoreType.DMA,
            pltpu.SemaphoreType.DMA,
            pltpu.SemaphoreType.DMA,
        ],
        compiler_params=pltpu.CompilerParams(
            use_tc_tiling_on_sc=True, needs_layout_passes=False,
            disable_bounds_checks=True),
    )

    out = emb(tok_t, table2)
    return jnp.transpose(out, (2, 0, 1))


# speedup vs baseline: 2.7434x; 1.4930x over previous
import math

import jax
import jax.numpy as jnp
from jax import lax
from jax.experimental import pallas as pl
from jax.experimental.pallas import tpu as pltpu
from jax.experimental.pallas import tpu_sc as plsc
import jax.experimental.layout
from jax.experimental.layout import Format, Layout

EMB_D = 64
SCALE = math.sqrt(EMB_D)

NUM_CORES = 2
NUM_SUBCORES = 16
NW = NUM_CORES * NUM_SUBCORES

CHUNK = 256
SEQ = 200
BATCH = 4096
BLKS_PER_S = BATCH // CHUNK
N_CHUNKS = SEQ * BLKS_PER_S
CPW = N_CHUNKS // NW


def _chunk_pos(c):
    return c // BLKS_PER_S, (c % BLKS_PER_S) * 2


def _emb_body(tok_hbm, table_hbm, out_hbm,
              tok0, tok1, rows0, rows1, outb0, outb1,
              tsem0, tsem1, gsem0, gsem1, osem0, osem1):
    w = lax.axis_index("s") * NUM_CORES + lax.axis_index("c")
    c0 = w * CPW

    toks = (tok0, tok1)
    rows = (rows0, rows1)
    outs = (outb0, outb1)
    tsems = (tsem0, tsem1)
    gsems = (gsem0, gsem1)
    osems = (osem0, osem1)

    def tok_copy(g, b):
        s, blk2 = _chunk_pos(c0 + g)
        return pltpu.make_async_copy(
            tok_hbm.at[s, pl.ds(blk2, 2)], toks[b], tsems[b])

    def gather_copy(b, j):
        return pltpu.make_async_copy(
            table_hbm.at[toks[b].at[j]],
            rows[b].at[pl.ds(j * 128, 128)], gsems[b])

    def out_copies(g, b):
        s, blk2 = _chunk_pos(c0 + g)
        cps = []
        for dg in range(8):
            for bgl in range(2):
                cps.append(pltpu.make_async_copy(
                    outs[b].at[pl.ds(dg * 8, 8), pl.ds(bgl * 128, 128)],
                    out_hbm.at[s, dg, blk2 + bgl], osems[b]))
        return cps

    def transpose_scale(b):
        iota = lax.iota(jnp.int32, 16)

        @plsc.parallel_loop(0, CHUNK // 16, unroll=8)
        def _(k):
            ridx = iota + k * 16
            for j in range(EMB_D):
                dvec = jnp.bitwise_and(iota + j, EMB_D - 1)
                v = plsc.load_gather(rows[b], [ridx, dvec])
                plsc.store_scatter(outs[b], [dvec, ridx], v * SCALE)

    cp = tok_copy(0, 0)
    cp.start()
    cp.wait()
    for j in range(2):
        gather_copy(0, j).start()
    tok_copy(1, 1).start()

    def iter_g(g, b):
        nb = 1 - b

        @pl.when(g >= 2)
        def _():
            for c in out_copies(g, b):
                c.wait()

        for j in range(2):
            gather_copy(b, j).wait()
        transpose_scale(b)
        for c in out_copies(g, b):
            c.start()

        @pl.when(g + 1 < CPW)
        def _():
            tok_copy(g + 1, nb).wait()
            for j in range(2):
                gather_copy(nb, j).start()

        @pl.when(g + 2 < CPW)
        def _():
            tok_copy(g + 2, b).start()

    def pair(i, carry):
        iter_g(2 * i, 0)
        iter_g(2 * i + 1, 1)
        return carry

    lax.fori_loop(0, CPW // 2, pair, 0)

    for c in out_copies(CPW - 2, 0):
        c.wait()
    for c in out_copies(CPW - 1, 1):
        c.wait()


def kernel(tokens, embedding_weight):
    bt, seq = tokens.shape
    table_d = jax.experimental.layout.with_layout_constraint(
        embedding_weight,
        Layout(major_to_minor=(0, 1), tiling=((8,),)))
    tok_t = tokens.T.reshape(seq, bt // 128, 128)

    mesh = plsc.VectorSubcoreMesh(core_axis_name="c", subcore_axis_name="s")
    emb = pl.kernel(
        _emb_body,
        mesh=mesh,
        out_type=jax.ShapeDtypeStruct((seq, 8, bt // 128, 8, 128),
                                      jnp.float32),
        scratch_types=[
            pltpu.VMEM((2, 128), jnp.int32),
            pltpu.VMEM((2, 128), jnp.int32),
            pltpu.VMEM((CHUNK, EMB_D), jnp.float32),
            pltpu.VMEM((CHUNK, EMB_D), jnp.float32),
            pltpu.VMEM((EMB_D, CHUNK), jnp.float32),
            pltpu.VMEM((EMB_D, CHUNK), jnp.float32),
            pltpu.SemaphoreType.DMA,
            pltpu.SemaphoreType.DMA,
            pltpu.SemaphoreType.DMA,
            pltpu.SemaphoreType.DMA,
            pltpu.SemaphoreType.DMA,
            pltpu.SemaphoreType.DMA,
        ],
        compiler_params=pltpu.CompilerParams(
            use_tc_tiling_on_sc=False, needs_layout_passes=False,
            disable_bounds_checks=True),
    )

    out5 = emb(tok_t, table_d)
    return out5.transpose(2, 4, 0, 1, 3).reshape(bt, seq, EMB_D)
